# baseline (speedup 1.0000x reference)
import functools

import jax
import jax.numpy as jnp
from jax import lax
from jax.experimental import pallas as pl
from jax.experimental.pallas import tpu as pltpu
from jax.experimental.pallas import tpu_sc as plsc

N = 10000
E = 320000
D = 128
NC = 2
NS = 16
NW = NC * NS
L = 16
NP = 10240
RPT = NP // NS
CH = 80
NR = E // NW // CH
SB = 25
NSC = NR // SB

_mesh = plsc.VectorSubcoreMesh(core_axis_name="c", subcore_axis_name="s")


def _zmm_body(f_ref, wfc_ref, z_ref):
    z_ref[...] = lax.dot_general(f_ref[...], wfc_ref[...],
                                 (((1,), (1,)), ((), ())),
                                 preferred_element_type=jnp.float32)


_zmm = pl.pallas_call(
    _zmm_body,
    out_shape=jax.ShapeDtypeStruct((N, D), jnp.float32),
)


def _attn_body(f_ref, wfc_ref, wat_ref, sl_ref, sr_ref):
    wat = wat_ref[...]
    wfc = wfc_ref[...]
    ul = lax.dot_general(wat[:, :D], wfc, (((1,), (0,)), ((), ())),
                         preferred_element_type=jnp.float32)
    ur = lax.dot_general(wat[:, D:], wfc, (((1,), (0,)), ((), ())),
                         preferred_element_type=jnp.float32)
    f = f_ref[...]
    sl_ref[...] = lax.dot_general(f, ul, (((1,), (1,)), ((), ())),
                                  preferred_element_type=jnp.float32)
    sr_ref[...] = lax.dot_general(f, ur, (((1,), (1,)), ((), ())),
                                  preferred_element_type=jnp.float32)


_attn = pl.pallas_call(
    _attn_body,
    out_shape=[
        jax.ShapeDtypeStruct((N, 1), jnp.float32),
        jax.ShapeDtypeStruct((N, 1), jnp.float32),
    ],
)


C1 = 128
NROW1 = E // C1
MR1 = NROW1 // NW + 1


@functools.partial(
    pl.kernel,
    out_type=[
        jax.ShapeDtypeStruct((E,), jnp.float32),
        jax.ShapeDtypeStruct((NC, NP), jnp.float32),
    ],
    mesh=_mesh,
    scratch_types=[
        pltpu.VMEM((MR1, C1), jnp.int32),
        pltpu.VMEM((MR1, C1), jnp.int32),
        pltpu.VMEM((MR1, C1), jnp.float32),
        pltpu.VMEM((MR1, C1), jnp.float32),
        pltpu.VMEM((MR1, C1), jnp.float32),
        pltpu.VMEM_SHARED((NP,), jnp.float32),
        pltpu.SemaphoreType.DMA,
        pltpu.SemaphoreType.DMA,
        pltpu.SemaphoreType.DMA,
    ],
)
def _sc_edge(sl_hbm, sr_hbm, src_hbm, dst_hbm, zvec_hbm,
             eexp_hbm, den_hbm,
             sblk, dblk, vl, vr, pblk, dacc, sem1, sem2, semsc):
    cid = lax.axis_index("c")
    sid = lax.axis_index("s")
    row0 = sid * RPT
    wid = sid * NC + cid
    r0 = (NROW1 * wid) // NW
    nrow = (NROW1 * (wid + 1)) // NW - r0
    wbase = r0 * C1

    @pl.loop(0, nrow)
    def _ld(j):
        pltpu.async_copy(src_hbm.at[pl.ds(wbase + j * C1, C1)], sblk.at[j],
                         sem1)
        pltpu.async_copy(dst_hbm.at[pl.ds(wbase + j * C1, C1)], dblk.at[j],
                         sem2)

    pltpu.sync_copy(zvec_hbm.at[pl.ds(row0, RPT)], dacc.at[pl.ds(row0, RPT)])
    plsc.subcore_barrier()

    @pl.loop(0, nrow)
    def _ldw(j):
        pltpu.make_async_copy(src_hbm.at[pl.ds(wbase + j * C1, C1)],
                              sblk.at[j], sem1).wait()
        pltpu.make_async_copy(dst_hbm.at[pl.ds(wbase + j * C1, C1)],
                              dblk.at[j], sem2).wait()

    @pl.loop(0, nrow)
    def _fire(j):
        pltpu.async_copy(sl_hbm.at[sblk.at[j]], vl.at[j], sem1)
        pltpu.async_copy(sr_hbm.at[dblk.at[j]], vr.at[j], sem2)

    @pl.loop(0, nrow)
    def _r(j):
        pltpu.make_async_copy(sl_hbm.at[sblk.at[j]], vl.at[j], sem1).wait()
        pltpu.make_async_copy(sr_hbm.at[dblk.at[j]], vr.at[j], sem2).wait()
        for k in range(C1 // L):
            s = pl.ds(k * L, L)
            a = vl[j, s] + vr[j, s]
            e = jnp.where(a >= 0.0, a, a * jnp.float32(0.01))
            pblk[j, s] = jnp.exp(e)
        pltpu.async_copy(pblk.at[j], dacc.at[sblk.at[j]], semsc, add=True)
        pltpu.async_copy(pblk.at[j], eexp_hbm.at[pl.ds(wbase + j * C1, C1)],
                         sem2)

    @pl.loop(0, nrow)
    def _dr(j):
        pltpu.make_async_copy(pblk.at[j], dacc.at[sblk.at[j]], semsc).wait()
        pltpu.make_async_copy(pblk.at[j],
                              eexp_hbm.at[pl.ds(wbase + j * C1, C1)],
                              sem2).wait()

    plsc.subcore_barrier()

    @pl.when(sid == 0)
    def _dump():
        pltpu.sync_copy(dacc, den_hbm.at[cid])


def _wscale_body(z_ref, d0_ref, d1_ref, w_ref):
    w_ref[...] = z_ref[...] * (1.0 / (d0_ref[...] + d1_ref[...]))


_wscale = pl.pallas_call(
    _wscale_body,
    grid=(5,),
    in_specs=[
        pl.BlockSpec((2000, D), lambda i: (i, 0)),
        pl.BlockSpec((2000, 1), lambda i: (i, 0)),
        pl.BlockSpec((2000, 1), lambda i: (i, 0)),
    ],
    out_specs=pl.BlockSpec((2000, D), lambda i: (i, 0)),
    out_shape=jax.ShapeDtypeStruct((N, D), jnp.float32),
)


@functools.partial(
    pl.kernel,
    out_type=jax.ShapeDtypeStruct((NC, NP, D), jnp.float32),
    mesh=_mesh,
    scratch_types=[
        pltpu.VMEM((SB * CH,), jnp.int32),
        pltpu.VMEM((SB, CH), jnp.int32),
        pltpu.VMEM((SB * CH,), jnp.float32),
        pltpu.VMEM((3, CH, D), jnp.float32),
        pltpu.VMEM_SHARED((NP, D), jnp.float32),
        pltpu.SemaphoreType.DMA,
        pltpu.SemaphoreType.DMA,
        pltpu.SemaphoreType.DMA,
    ],
)
def _sc_agg(w_hbm, eexp_hbm, src_hbm, dst_hbm, zmat_hbm,
            hp_hbm,
            sblk, dblk, pblk, zr, hacc, semz, semsc, sem1):
    cid = lax.axis_index("c")
    sid = lax.axis_index("s")
    row0 = sid * RPT
    wid = sid * NC + cid
    wbase = wid * NR * CH

    pltpu.sync_copy(zmat_hbm.at[pl.ds(row0, RPT)], hacc.at[pl.ds(row0, RPT)])
    plsc.subcore_barrier()

    @pl.loop(0, NSC)
    def _super(sc):
        base = wbase + sc * SB * CH
        cs = pltpu.async_copy(src_hbm.at[pl.ds(base, SB * CH)], sblk, sem1)
        cp = pltpu.async_copy(eexp_hbm.at[pl.ds(base, SB * CH)], pblk, semz)

        @pl.loop(0, SB)
        def _ldd(jj):
            pltpu.async_copy(dst_hbm.at[pl.ds(base + jj * CH, CH)],
                             dblk.at[jj], semsc)

        cs.wait()
        cp.wait()

        @pl.loop(0, SB)
        def _ldw(jj):
            pltpu.make_async_copy(dst_hbm.at[pl.ds(base + jj * CH, CH)],
                                  dblk.at[jj], semsc).wait()

        pltpu.async_copy(w_hbm.at[sblk.at[pl.ds(0, CH)]], zr.at[0], semz)

        @pl.loop(0, SB)
        def _row(jj):
            b = lax.rem(jj, 3)

            @pl.when(jj >= 2)
            def _drain_scatter():
                bd = lax.rem(jj + 1, 3)
                pltpu.make_async_copy(zr.at[bd], hacc.at[dblk.at[jj - 2]],
                                      semsc).wait()

            @pl.when(jj + 1 < SB)
            def _issue_gather():
                bn = lax.rem(jj + 1, 3)
                pltpu.async_copy(
                    w_hbm.at[sblk.at[pl.ds((jj + 1) * CH, CH)]],
                    zr.at[bn], semz)

            pltpu.make_async_copy(w_hbm.at[sblk.at[pl.ds(jj * CH, CH)]],
                                  zr.at[b], semz).wait()

            for g in range(CH // L):
                a16 = pblk[pl.ds(jj * CH + g * L, L)]
                for i in range(L):
                    r = g * L + i
                    for k in range(D // L):
                        s = pl.ds(k * L, L)
                        zr[b, r, s] = zr[b, r, s] * a16[i]

            pltpu.async_copy(zr.at[b], hacc.at[dblk.at[jj]], semsc, add=True)

        pltpu.make_async_copy(zr.at[(SB - 2) % 3], hacc.at[dblk.at[SB - 2]],
                              semsc).wait()
        pltpu.make_async_copy(zr.at[(SB - 1) % 3], hacc.at[dblk.at[SB - 1]],
                              semsc).wait()

    plsc.subcore_barrier()
    pltpu.sync_copy(hacc.at[pl.ds(row0, RPT)],
                    hp_hbm.at[cid, pl.ds(row0, RPT)])


def _combine_body(a_ref, b_ref, o_ref):
    o_ref[...] = a_ref[...] + b_ref[...]


_combine = pl.pallas_call(
    _combine_body,
    grid=(5,),
    in_specs=[
        pl.BlockSpec((2000, D), lambda i: (i, 0)),
        pl.BlockSpec((2000, D), lambda i: (i, 0)),
    ],
    out_specs=pl.BlockSpec((2000, D), lambda i: (i, 0)),
    out_shape=jax.ShapeDtypeStruct((N, D), jnp.float32),
)


def kernel(feature, edge_index, W_fc, W_attn):
    src = edge_index[0].astype(jnp.int32)
    dst = edge_index[1].astype(jnp.int32)
    sl, sr = _attn(feature, W_fc, W_attn)
    z = _zmm(feature, W_fc)
    sl = sl.reshape(N)
    sr = sr.reshape(N)
    zvec = jnp.zeros((NP,), jnp.float32)
    zmat = jnp.zeros((NP, D), jnp.float32)
    eexp, den = _sc_edge(sl, sr, src, dst, zvec)
    d0 = den[0, :N].reshape(N, 1)
    d1 = den[1, :N].reshape(N, 1)
    w = _wscale(z, d0, d1)
    hp = _sc_agg(w, eexp, src, dst, zmat)
    return _combine(hp[0, :N], hp[1, :N])


# speedup vs baseline: 1.0182x; 1.0182x over previous
import functools

import jax
import jax.numpy as jnp
from jax import lax
from jax.experimental import pallas as pl
from jax.experimental.pallas import tpu as pltpu
from jax.experimental.pallas import tpu_sc as plsc

N = 10000
E = 320000
D = 128
NC = 2
NS = 16
NW = NC * NS
L = 16
NP = 10240
RPT = NP // NS
CH = 80
NR = E // NW // CH
SB = 25
NSC = NR // SB

_mesh = plsc.VectorSubcoreMesh(core_axis_name="c", subcore_axis_name="s")


def _zmm_body(f_ref, wfc_ref, z_ref):
    z_ref[...] = lax.dot_general(f_ref[...], wfc_ref[...],
                                 (((1,), (1,)), ((), ())),
                                 preferred_element_type=jnp.float32)


_zmm = pl.pallas_call(
    _zmm_body,
    out_shape=jax.ShapeDtypeStruct((N, D), jnp.float32),
)


def _attn_body(f_ref, wfc_ref, wat_ref, sl_ref, sr_ref):
    wat = wat_ref[...]
    wfc = wfc_ref[...]
    ul = lax.dot_general(wat[:, :D], wfc, (((1,), (0,)), ((), ())),
                         preferred_element_type=jnp.float32)
    ur = lax.dot_general(wat[:, D:], wfc, (((1,), (0,)), ((), ())),
                         preferred_element_type=jnp.float32)
    f = f_ref[...]
    sl_ref[...] = lax.dot_general(f, ul, (((1,), (1,)), ((), ())),
                                  preferred_element_type=jnp.float32)
    sr_ref[...] = lax.dot_general(f, ur, (((1,), (1,)), ((), ())),
                                  preferred_element_type=jnp.float32)


_attn = pl.pallas_call(
    _attn_body,
    out_shape=[
        jax.ShapeDtypeStruct((N, 1), jnp.float32),
        jax.ShapeDtypeStruct((N, 1), jnp.float32),
    ],
)


@functools.partial(
    pl.kernel,
    out_type=[
        jax.ShapeDtypeStruct((E,), jnp.float32),
        jax.ShapeDtypeStruct((NC, NP), jnp.float32),
    ],
    mesh=_mesh,
    scratch_types=[
        pltpu.VMEM((NR, CH), jnp.int32),
        pltpu.VMEM((NR, CH), jnp.int32),
        pltpu.VMEM((NR, CH), jnp.float32),
        pltpu.VMEM((NR, CH), jnp.float32),
        pltpu.VMEM((NR, CH), jnp.float32),
        pltpu.VMEM_SHARED((NP,), jnp.float32),
        pltpu.SemaphoreType.DMA,
        pltpu.SemaphoreType.DMA,
        pltpu.SemaphoreType.DMA,
    ],
)
def _sc_edge(sl_hbm, sr_hbm, src_hbm, dst_hbm, zvec_hbm,
             eexp_hbm, den_hbm,
             sblk, dblk, vl, vr, pblk, dacc, sem1, sem2, semsc):
    cid = lax.axis_index("c")
    sid = lax.axis_index("s")
    row0 = sid * RPT
    wid = sid * NC + cid
    wbase = wid * NR * CH

    @pl.loop(0, NR)
    def _ld(j):
        pltpu.async_copy(src_hbm.at[pl.ds(wbase + j * CH, CH)], sblk.at[j],
                         sem1)
        pltpu.async_copy(dst_hbm.at[pl.ds(wbase + j * CH, CH)], dblk.at[j],
                         sem2)

    pltpu.sync_copy(zvec_hbm.at[pl.ds(row0, RPT)], dacc.at[pl.ds(row0, RPT)])
    plsc.subcore_barrier()

    @pl.loop(0, NR)
    def _ldw(j):
        pltpu.make_async_copy(src_hbm.at[pl.ds(wbase + j * CH, CH)],
                              sblk.at[j], sem1).wait()
        pltpu.make_async_copy(dst_hbm.at[pl.ds(wbase + j * CH, CH)],
                              dblk.at[j], sem2).wait()

    @pl.loop(0, NR)
    def _fire(j):
        pltpu.async_copy(sl_hbm.at[sblk.at[j]], vl.at[j], sem1)
        pltpu.async_copy(sr_hbm.at[dblk.at[j]], vr.at[j], sem2)

    @pl.loop(0, NR)
    def _r(j):
        pltpu.make_async_copy(sl_hbm.at[sblk.at[j]], vl.at[j], sem1).wait()
        pltpu.make_async_copy(sr_hbm.at[dblk.at[j]], vr.at[j], sem2).wait()
        for k in range(CH // L):
            s = pl.ds(k * L, L)
            a = vl[j, s] + vr[j, s]
            e = jnp.where(a >= 0.0, a, a * jnp.float32(0.01))
            pblk[j, s] = jnp.exp(e)
        pltpu.async_copy(pblk.at[j], dacc.at[sblk.at[j]], semsc, add=True)
        pltpu.async_copy(pblk.at[j], eexp_hbm.at[pl.ds(wbase + j * CH, CH)],
                         sem2)

    @pl.loop(0, NR)
    def _dr(j):
        pltpu.make_async_copy(pblk.at[j], dacc.at[sblk.at[j]], semsc).wait()
        pltpu.make_async_copy(pblk.at[j],
                              eexp_hbm.at[pl.ds(wbase + j * CH, CH)],
                              sem2).wait()

    plsc.subcore_barrier()

    @pl.when(sid == 0)
    def _dump():
        pltpu.sync_copy(dacc, den_hbm.at[cid])


def _wscale_body(z_ref, d0_ref, d1_ref, w_ref):
    w_ref[...] = z_ref[...] * (1.0 / (d0_ref[...] + d1_ref[...]))


_wscale = pl.pallas_call(
    _wscale_body,
    grid=(5,),
    in_specs=[
        pl.BlockSpec((2000, D), lambda i: (i, 0)),
        pl.BlockSpec((2000, 1), lambda i: (i, 0)),
        pl.BlockSpec((2000, 1), lambda i: (i, 0)),
    ],
    out_specs=pl.BlockSpec((2000, D), lambda i: (i, 0)),
    out_shape=jax.ShapeDtypeStruct((N, D), jnp.float32),
)


@functools.partial(
    pl.kernel,
    out_type=jax.ShapeDtypeStruct((NC, NP, D), jnp.float32),
    mesh=_mesh,
    scratch_types=[
        pltpu.VMEM((SB * CH,), jnp.int32),
        pltpu.VMEM((SB, CH), jnp.int32),
        pltpu.VMEM((SB * CH,), jnp.float32),
        pltpu.VMEM((3, CH, D), jnp.float32),
        pltpu.VMEM_SHARED((NP, D), jnp.float32),
        pltpu.SemaphoreType.DMA,
        pltpu.SemaphoreType.DMA,
        pltpu.SemaphoreType.DMA,
    ],
)
def _sc_agg(w_hbm, eexp_hbm, src_hbm, dst_hbm, zmat_hbm,
            hp_hbm,
            sblk, dblk, pblk, zr, hacc, semz, semsc, sem1):
    cid = lax.axis_index("c")
    sid = lax.axis_index("s")
    row0 = sid * RPT
    wid = sid * NC + cid
    wbase = wid * NR * CH

    pltpu.sync_copy(zmat_hbm.at[pl.ds(row0, RPT)], hacc.at[pl.ds(row0, RPT)])
    plsc.subcore_barrier()

    @pl.loop(0, NSC)
    def _super(sc):
        base = wbase + sc * SB * CH
        cs = pltpu.async_copy(src_hbm.at[pl.ds(base, SB * CH)], sblk, sem1)
        cp = pltpu.async_copy(eexp_hbm.at[pl.ds(base, SB * CH)], pblk, semz)

        @pl.loop(0, SB)
        def _ldd(jj):
            pltpu.async_copy(dst_hbm.at[pl.ds(base + jj * CH, CH)],
                             dblk.at[jj], semsc)

        cs.wait()
        cp.wait()

        @pl.loop(0, SB)
        def _ldw(jj):
            pltpu.make_async_copy(dst_hbm.at[pl.ds(base + jj * CH, CH)],
                                  dblk.at[jj], semsc).wait()

        pltpu.async_copy(w_hbm.at[sblk.at[pl.ds(0, CH)]], zr.at[0], semz)

        @pl.loop(0, SB)
        def _row(jj):
            b = lax.rem(jj, 3)

            @pl.when(jj >= 2)
            def _drain_scatter():
                bd = lax.rem(jj + 1, 3)
                pltpu.make_async_copy(zr.at[bd], hacc.at[dblk.at[jj - 2]],
                                      semsc).wait()

            @pl.when(jj + 1 < SB)
            def _issue_gather():
                bn = lax.rem(jj + 1, 3)
                pltpu.async_copy(
                    w_hbm.at[sblk.at[pl.ds((jj + 1) * CH, CH)]],
                    zr.at[bn], semz)

            pltpu.make_async_copy(w_hbm.at[sblk.at[pl.ds(jj * CH, CH)]],
                                  zr.at[b], semz).wait()

            for g in range(CH // L):
                a16 = pblk[pl.ds(jj * CH + g * L, L)]
                for i in range(L):
                    r = g * L + i
                    for k in range(D // L):
                        s = pl.ds(k * L, L)
                        zr[b, r, s] = zr[b, r, s] * a16[i]

            pltpu.async_copy(zr.at[b], hacc.at[dblk.at[jj]], semsc, add=True)

        pltpu.make_async_copy(zr.at[(SB - 2) % 3], hacc.at[dblk.at[SB - 2]],
                              semsc).wait()
        pltpu.make_async_copy(zr.at[(SB - 1) % 3], hacc.at[dblk.at[SB - 1]],
                              semsc).wait()

    plsc.subcore_barrier()
    pltpu.sync_copy(hacc.at[pl.ds(row0, RPT)],
                    hp_hbm.at[cid, pl.ds(row0, RPT)])


def _combine_body(a_ref, b_ref, o_ref):
    o_ref[...] = a_ref[...] + b_ref[...]


_combine = pl.pallas_call(
    _combine_body,
    grid=(5,),
    in_specs=[
        pl.BlockSpec((2000, D), lambda i: (i, 0)),
        pl.BlockSpec((2000, D), lambda i: (i, 0)),
    ],
    out_specs=pl.BlockSpec((2000, D), lambda i: (i, 0)),
    out_shape=jax.ShapeDtypeStruct((N, D), jnp.float32),
)


def kernel(feature, edge_index, W_fc, W_attn):
    src = edge_index[0].astype(jnp.int32)
    dst = edge_index[1].astype(jnp.int32)
    sl, sr = _attn(feature, W_fc, W_attn)
    z = _zmm(feature, W_fc)
    sl = sl.reshape(N)
    sr = sr.reshape(N)
    zvec = jnp.zeros((NP,), jnp.float32)
    zmat = jnp.zeros((NP, D), jnp.float32)
    eexp, den = _sc_edge(sl, sr, src, dst, zvec)
    d0 = den[0, :N].reshape(N, 1)
    d1 = den[1, :N].reshape(N, 1)
    w = _wscale(z, d0, d1)
    hp = _sc_agg(w, eexp, src, dst, zmat)
    return _combine(hp[0, :N], hp[1, :N])


# speedup vs baseline: 1.0765x; 1.0572x over previous
import functools

import jax
import jax.numpy as jnp
from jax import lax
from jax.experimental import pallas as pl
from jax.experimental.pallas import tpu as pltpu
from jax.experimental.pallas import tpu_sc as plsc

N = 10000
E = 320000
D = 128
NC = 2
NS = 16
NW = NC * NS
L = 16
NP = 10240
RPT = NP // NS
CH = 80
NR = E // NW // CH
SB = 25
NSC = NR // SB
WB = 64
NWB = RPT // WB

_mesh = plsc.VectorSubcoreMesh(core_axis_name="c", subcore_axis_name="s")


def _prep_body(f_ref, wfc_ref, wat_ref, z_ref, sl_ref, sr_ref):
    z = lax.dot_general(f_ref[...], wfc_ref[...], (((1,), (1,)), ((), ())),
                        preferred_element_type=jnp.float32)
    z_ref[...] = z
    wat = wat_ref[...]
    sl_ref[...] = lax.dot_general(z, wat[:, :D], (((1,), (1,)), ((), ())),
                                  preferred_element_type=jnp.float32)
    sr_ref[...] = lax.dot_general(z, wat[:, D:], (((1,), (1,)), ((), ())),
                                  preferred_element_type=jnp.float32)


_prep = pl.pallas_call(
    _prep_body,
    out_shape=[
        jax.ShapeDtypeStruct((NP, D), jnp.float32),
        jax.ShapeDtypeStruct((NP, 1), jnp.float32),
        jax.ShapeDtypeStruct((NP, 1), jnp.float32),
    ],
)


@functools.partial(
    pl.kernel,
    out_type=[
        jax.ShapeDtypeStruct((E,), jnp.float32),
        jax.ShapeDtypeStruct((NC, NP), jnp.float32),
    ],
    mesh=_mesh,
    scratch_types=[
        pltpu.VMEM((NR, CH), jnp.int32),
        pltpu.VMEM((NR, CH), jnp.int32),
        pltpu.VMEM((NR, CH), jnp.float32),
        pltpu.VMEM((NR, CH), jnp.float32),
        pltpu.VMEM((NR, CH), jnp.float32),
        pltpu.VMEM_SHARED((NP,), jnp.float32),
        pltpu.SemaphoreType.DMA,
        pltpu.SemaphoreType.DMA,
        pltpu.SemaphoreType.DMA,
    ],
)
def _sc_edge(sl_hbm, sr_hbm, src_hbm, dst_hbm, zvec_hbm,
             eexp_hbm, den_hbm,
             sblk, dblk, vl, vr, pblk, dacc, sem1, sem2, semsc):
    cid = lax.axis_index("c")
    sid = lax.axis_index("s")
    row0 = sid * RPT
    wid = sid * NC + cid
    wbase = wid * NR * CH

    @pl.loop(0, NR)
    def _ld(j):
        pltpu.async_copy(src_hbm.at[pl.ds(wbase + j * CH, CH)], sblk.at[j],
                         sem1)
        pltpu.async_copy(dst_hbm.at[pl.ds(wbase + j * CH, CH)], dblk.at[j],
                         sem2)

    pltpu.sync_copy(zvec_hbm.at[pl.ds(row0, RPT)], dacc.at[pl.ds(row0, RPT)])
    plsc.subcore_barrier()

    @pl.loop(0, NR)
    def _ldw(j):
        pltpu.make_async_copy(src_hbm.at[pl.ds(wbase + j * CH, CH)],
                              sblk.at[j], sem1).wait()
        pltpu.make_async_copy(dst_hbm.at[pl.ds(wbase + j * CH, CH)],
                              dblk.at[j], sem2).wait()

    @pl.loop(0, NR)
    def _fire(j):
        pltpu.async_copy(sl_hbm.at[sblk.at[j]], vl.at[j], sem1)
        pltpu.async_copy(sr_hbm.at[dblk.at[j]], vr.at[j], sem2)

    @pl.loop(0, NR)
    def _r(j):
        pltpu.make_async_copy(sl_hbm.at[sblk.at[j]], vl.at[j], sem1).wait()
        pltpu.make_async_copy(sr_hbm.at[dblk.at[j]], vr.at[j], sem2).wait()
        for k in range(CH // L):
            s = pl.ds(k * L, L)
            a = vl[j, s] + vr[j, s]
            e = jnp.where(a >= 0.0, a, a * jnp.float32(0.01))
            pblk[j, s] = jnp.exp(e)
        pltpu.async_copy(pblk.at[j], dacc.at[sblk.at[j]], semsc, add=True)
        pltpu.async_copy(pblk.at[j], eexp_hbm.at[pl.ds(wbase + j * CH, CH)],
                         sem2)

    @pl.loop(0, NR)
    def _dr(j):
        pltpu.make_async_copy(pblk.at[j], dacc.at[sblk.at[j]], semsc).wait()
        pltpu.make_async_copy(pblk.at[j],
                              eexp_hbm.at[pl.ds(wbase + j * CH, CH)],
                              sem2).wait()

    plsc.subcore_barrier()

    @pl.when(sid == 0)
    def _dump():
        pltpu.sync_copy(dacc, den_hbm.at[cid])


@functools.partial(
    pl.kernel,
    out_type=[
        jax.ShapeDtypeStruct((NC * NP, D), jnp.float32),
        jax.ShapeDtypeStruct((NC, NP, D), jnp.float32),
    ],
    mesh=_mesh,
    scratch_types=[
        pltpu.VMEM((SB * CH,), jnp.int32),
        pltpu.VMEM((SB, CH), jnp.int32),
        pltpu.VMEM((SB * CH,), jnp.float32),
        pltpu.VMEM((3, CH, D), jnp.float32),
        pltpu.VMEM((WB, D), jnp.float32),
        pltpu.VMEM((WB,), jnp.float32),
        pltpu.VMEM((WB,), jnp.float32),
        pltpu.VMEM_SHARED((NP, D), jnp.float32),
        pltpu.SemaphoreType.DMA,
        pltpu.SemaphoreType.DMA,
        pltpu.SemaphoreType.DMA,
    ],
)
def _sc_agg(z_hbm, eexp_hbm, d0_hbm, d1_hbm, src_hbm, dst_hbm, zmat_hbm,
            wtab_hbm, hp_hbm,
            sblk, dblk, pblk, zr, zw, d0v, d1v, hacc, semz, semsc, sem1):
    cid = lax.axis_index("c")
    sid = lax.axis_index("s")
    row0 = sid * RPT
    wid = sid * NC + cid
    wbase = wid * NR * CH
    cbase = cid * NP

    pltpu.sync_copy(zmat_hbm.at[pl.ds(row0, RPT)], hacc.at[pl.ds(row0, RPT)])

    @pl.loop(0, NWB)
    def _wblk(sb):
        r0 = row0 + sb * WB
        pltpu.sync_copy(z_hbm.at[pl.ds(r0, WB)], zw)
        pltpu.sync_copy(d0_hbm.at[pl.ds(r0, WB)], d0v)
        pltpu.sync_copy(d1_hbm.at[pl.ds(r0, WB)], d1v)
        for g in range(WB // L):
            rec = 1.0 / (d0v[pl.ds(g * L, L)] + d1v[pl.ds(g * L, L)])
            for i in range(L):
                r = g * L + i
                for k in range(D // L):
                    s = pl.ds(k * L, L)
                    zw[r, s] = zw[r, s] * rec[i]
        pltpu.sync_copy(zw, wtab_hbm.at[pl.ds(cbase + r0, WB)])

    plsc.subcore_barrier()

    @pl.loop(0, NSC)
    def _super(sc):
        base = wbase + sc * SB * CH
        cs = pltpu.async_copy(src_hbm.at[pl.ds(base, SB * CH)], sblk, sem1)
        cp = pltpu.async_copy(eexp_hbm.at[pl.ds(base, SB * CH)], pblk, semz)

        @pl.loop(0, SB)
        def _ldd(jj):
            pltpu.async_copy(dst_hbm.at[pl.ds(base + jj * CH, CH)],
                             dblk.at[jj], semsc)

        cs.wait()
        cp.wait()

        @pl.loop(0, SB * CH // L)
        def _adj(i):
            s = pl.ds(i * L, L)
            sblk[s] = sblk[s] + cbase

        @pl.loop(0, SB)
        def _ldw(jj):
            pltpu.make_async_copy(dst_hbm.at[pl.ds(base + jj * CH, CH)],
                                  dblk.at[jj], semsc).wait()

        pltpu.async_copy(wtab_hbm.at[sblk.at[pl.ds(0, CH)]], zr.at[0], semz)

        @pl.loop(0, SB)
        def _row(jj):
            b = lax.rem(jj, 3)

            @pl.when(jj >= 2)
            def _drain_scatter():
                bd = lax.rem(jj + 1, 3)
                pltpu.make_async_copy(zr.at[bd], hacc.at[dblk.at[jj - 2]],
                                      semsc).wait()

            @pl.when(jj + 1 < SB)
            def _issue_gather():
                bn = lax.rem(jj + 1, 3)
                pltpu.async_copy(
                    wtab_hbm.at[sblk.at[pl.ds((jj + 1) * CH, CH)]],
                    zr.at[bn], semz)

            pltpu.make_async_copy(wtab_hbm.at[sblk.at[pl.ds(jj * CH, CH)]],
                                  zr.at[b], semz).wait()

            for g in range(CH // L):
                a16 = pblk[pl.ds(jj * CH + g * L, L)]
                for i in range(L):
                    r = g * L + i
                    for k in range(D // L):
                        s = pl.ds(k * L, L)
                        zr[b, r, s] = zr[b, r, s] * a16[i]

            pltpu.async_copy(zr.at[b], hacc.at[dblk.at[jj]], semsc, add=True)

        pltpu.make_async_copy(zr.at[(SB - 2) % 3], hacc.at[dblk.at[SB - 2]],
                              semsc).wait()
        pltpu.make_async_copy(zr.at[(SB - 1) % 3], hacc.at[dblk.at[SB - 1]],
                              semsc).wait()

    plsc.subcore_barrier()
    pltpu.sync_copy(hacc.at[pl.ds(row0, RPT)],
                    hp_hbm.at[cid, pl.ds(row0, RPT)])


def _combine_body(a_ref, b_ref, o_ref):
    o_ref[...] = a_ref[...] + b_ref[...]


_combine = pl.pallas_call(
    _combine_body,
    grid=(5,),
    in_specs=[
        pl.BlockSpec((2000, D), lambda i: (i, 0)),
        pl.BlockSpec((2000, D), lambda i: (i, 0)),
    ],
    out_specs=pl.BlockSpec((2000, D), lambda i: (i, 0)),
    out_shape=jax.ShapeDtypeStruct((N, D), jnp.float32),
)


def kernel(feature, edge_index, W_fc, W_attn):
    src = edge_index[0].astype(jnp.int32)
    dst = edge_index[1].astype(jnp.int32)
    fpad = jnp.pad(feature, ((0, NP - N), (0, 0)))
    z, sl, sr = _prep(fpad, W_fc, W_attn)
    sl = sl.reshape(NP)
    sr = sr.reshape(NP)
    zvec = jnp.zeros((NP,), jnp.float32)
    zmat = jnp.zeros((NP, D), jnp.float32)
    eexp, den = _sc_edge(sl, sr, src, dst, zvec)
    _, hp = _sc_agg(z, eexp, den[0], den[1], src, dst, zmat)
    return _combine(hp[0, :N], hp[1, :N])


# speedup vs baseline: 1.1106x; 1.0317x over previous
import functools

import jax
import jax.numpy as jnp
from jax import lax
from jax.experimental import pallas as pl
from jax.experimental.pallas import tpu as pltpu
from jax.experimental.pallas import tpu_sc as plsc

N = 10000
E = 320000
D = 128
NC = 2
NS = 16
NW = NC * NS
L = 16
NP = 10240
RPT = NP // NS
CH = 80
NR = E // NW // CH
SB = 25
NSC = NR // SB

_mesh = plsc.VectorSubcoreMesh(core_axis_name="c", subcore_axis_name="s")


def _prep_body(f_ref, wfc_ref, wat_ref, z_ref, sl_ref, sr_ref):
    z = lax.dot_general(f_ref[...], wfc_ref[...], (((1,), (1,)), ((), ())),
                        preferred_element_type=jnp.float32)
    z_ref[...] = z
    wat = wat_ref[...]
    sl_ref[...] = lax.dot_general(z, wat[:, :D], (((1,), (1,)), ((), ())),
                                  preferred_element_type=jnp.float32)
    sr_ref[...] = lax.dot_general(z, wat[:, D:], (((1,), (1,)), ((), ())),
                                  preferred_element_type=jnp.float32)


_prep = pl.pallas_call(
    _prep_body,
    out_shape=[
        jax.ShapeDtypeStruct((N, D), jnp.float32),
        jax.ShapeDtypeStruct((N, 1), jnp.float32),
        jax.ShapeDtypeStruct((N, 1), jnp.float32),
    ],
)


@functools.partial(
    pl.kernel,
    out_type=[
        jax.ShapeDtypeStruct((E,), jnp.float32),
        jax.ShapeDtypeStruct((NC, NP), jnp.float32),
    ],
    mesh=_mesh,
    scratch_types=[
        pltpu.VMEM((NR, CH), jnp.int32),
        pltpu.VMEM((NR, CH), jnp.int32),
        pltpu.VMEM((NR, CH), jnp.float32),
        pltpu.VMEM((NR, CH), jnp.float32),
        pltpu.VMEM((NR, CH), jnp.float32),
        pltpu.VMEM_SHARED((NP,), jnp.float32),
        pltpu.SemaphoreType.DMA,
        pltpu.SemaphoreType.DMA,
        pltpu.SemaphoreType.DMA,
    ],
)
def _sc_edge(sl_hbm, sr_hbm, src_hbm, dst_hbm, zvec_hbm,
             eexp_hbm, den_hbm,
             sblk, dblk, vl, vr, pblk, dacc, sem1, sem2, semsc):
    cid = lax.axis_index("c")
    sid = lax.axis_index("s")
    row0 = sid * RPT
    wid = sid * NC + cid
    wbase = wid * NR * CH

    @pl.loop(0, NR)
    def _ld(j):
        pltpu.async_copy(src_hbm.at[pl.ds(wbase + j * CH, CH)], sblk.at[j],
                         sem1)
        pltpu.async_copy(dst_hbm.at[pl.ds(wbase + j * CH, CH)], dblk.at[j],
                         sem2)

    pltpu.sync_copy(zvec_hbm.at[pl.ds(row0, RPT)], dacc.at[pl.ds(row0, RPT)])
    plsc.subcore_barrier()

    @pl.loop(0, NR)
    def _ldw(j):
        pltpu.make_async_copy(src_hbm.at[pl.ds(wbase + j * CH, CH)],
                              sblk.at[j], sem1).wait()
        pltpu.make_async_copy(dst_hbm.at[pl.ds(wbase + j * CH, CH)],
                              dblk.at[j], sem2).wait()

    @pl.loop(0, NR)
    def _fire(j):
        pltpu.async_copy(sl_hbm.at[sblk.at[j]], vl.at[j], sem1)
        pltpu.async_copy(sr_hbm.at[dblk.at[j]], vr.at[j], sem2)

    @pl.loop(0, NR)
    def _r(j):
        pltpu.make_async_copy(sl_hbm.at[sblk.at[j]], vl.at[j], sem1).wait()
        pltpu.make_async_copy(sr_hbm.at[dblk.at[j]], vr.at[j], sem2).wait()
        for k in range(CH // L):
            s = pl.ds(k * L, L)
            a = vl[j, s] + vr[j, s]
            e = jnp.where(a >= 0.0, a, a * jnp.float32(0.01))
            pblk[j, s] = jnp.exp(e)
        pltpu.async_copy(pblk.at[j], dacc.at[sblk.at[j]], semsc, add=True)
        pltpu.async_copy(pblk.at[j], eexp_hbm.at[pl.ds(wbase + j * CH, CH)],
                         sem2)

    @pl.loop(0, NR)
    def _dr(j):
        pltpu.make_async_copy(pblk.at[j], dacc.at[sblk.at[j]], semsc).wait()
        pltpu.make_async_copy(pblk.at[j],
                              eexp_hbm.at[pl.ds(wbase + j * CH, CH)],
                              sem2).wait()

    plsc.subcore_barrier()

    @pl.when(sid == 0)
    def _dump():
        pltpu.sync_copy(dacc, den_hbm.at[cid])


@functools.partial(
    pl.kernel,
    out_type=[
        jax.ShapeDtypeStruct((NC * NP,), jnp.float32),
        jax.ShapeDtypeStruct((NC, NP, D), jnp.float32),
    ],
    mesh=_mesh,
    scratch_types=[
        pltpu.VMEM((SB * CH,), jnp.int32),
        pltpu.VMEM((SB * CH,), jnp.int32),
        pltpu.VMEM((SB, CH), jnp.int32),
        pltpu.VMEM((SB * CH,), jnp.float32),
        pltpu.VMEM((SB * CH,), jnp.float32),
        pltpu.VMEM((3, CH, D), jnp.float32),
        pltpu.VMEM((RPT,), jnp.float32),
        pltpu.VMEM((RPT,), jnp.float32),
        pltpu.VMEM_SHARED((NP, D), jnp.float32),
        pltpu.SemaphoreType.DMA,
        pltpu.SemaphoreType.DMA,
        pltpu.SemaphoreType.DMA,
    ],
)
def _sc_agg(z_hbm, eexp_hbm, d0_hbm, d1_hbm, src_hbm, dst_hbm, zmat_hbm,
            rtab_hbm, hp_hbm,
            sblk, rsidx, dblk, pblk, rblk, zr, rp0, rp1, hacc,
            semz, semsc, sem1):
    cid = lax.axis_index("c")
    sid = lax.axis_index("s")
    row0 = sid * RPT
    wid = sid * NC + cid
    wbase = wid * NR * CH
    cbase = cid * NP

    pltpu.sync_copy(d0_hbm.at[pl.ds(row0, RPT)], rp0)
    pltpu.sync_copy(d1_hbm.at[pl.ds(row0, RPT)], rp1)

    @pl.loop(0, RPT // L)
    def _rinv(i):
        s = pl.ds(i * L, L)
        rp0[s] = 1.0 / (rp0[s] + rp1[s])

    pltpu.sync_copy(rp0, rtab_hbm.at[pl.ds(cbase + row0, RPT)])
    pltpu.sync_copy(zmat_hbm.at[pl.ds(row0, RPT)], hacc.at[pl.ds(row0, RPT)])
    plsc.subcore_barrier()

    @pl.loop(0, NSC)
    def _super(sc):
        base = wbase + sc * SB * CH
        cs = pltpu.async_copy(src_hbm.at[pl.ds(base, SB * CH)], sblk, sem1)
        cp = pltpu.async_copy(eexp_hbm.at[pl.ds(base, SB * CH)], pblk, semz)

        @pl.loop(0, SB)
        def _ldd(jj):
            pltpu.async_copy(dst_hbm.at[pl.ds(base + jj * CH, CH)],
                             dblk.at[jj], semsc)

        cs.wait()
        cp.wait()

        @pl.loop(0, SB * CH // L)
        def _adj(i):
            s = pl.ds(i * L, L)
            rsidx[s] = sblk[s] + cbase

        @pl.loop(0, SB)
        def _rfire(jj):
            pltpu.async_copy(rtab_hbm.at[rsidx.at[pl.ds(jj * CH, CH)]],
                             rblk.at[pl.ds(jj * CH, CH)], sem1)

        @pl.loop(0, SB)
        def _ldw(jj):
            pltpu.make_async_copy(dst_hbm.at[pl.ds(base + jj * CH, CH)],
                                  dblk.at[jj], semsc).wait()

        pltpu.async_copy(z_hbm.at[sblk.at[pl.ds(0, CH)]], zr.at[0], semz)

        @pl.loop(0, SB)
        def _row(jj):
            b = lax.rem(jj, 3)

            @pl.when(jj >= 2)
            def _drain_scatter():
                bd = lax.rem(jj + 1, 3)
                pltpu.make_async_copy(zr.at[bd], hacc.at[dblk.at[jj - 2]],
                                      semsc).wait()

            @pl.when(jj + 1 < SB)
            def _issue_gather():
                bn = lax.rem(jj + 1, 3)
                pltpu.async_copy(
                    z_hbm.at[sblk.at[pl.ds((jj + 1) * CH, CH)]],
                    zr.at[bn], semz)

            pltpu.make_async_copy(rtab_hbm.at[rsidx.at[pl.ds(jj * CH, CH)]],
                                  rblk.at[pl.ds(jj * CH, CH)], sem1).wait()
            pltpu.make_async_copy(z_hbm.at[sblk.at[pl.ds(jj * CH, CH)]],
                                  zr.at[b], semz).wait()

            for g in range(CH // L):
                s0 = pl.ds(jj * CH + g * L, L)
                a16 = pblk[s0] * rblk[s0]
                for i in range(L):
                    r = g * L + i
                    for k in range(D // L):
                        s = pl.ds(k * L, L)
                        zr[b, r, s] = zr[b, r, s] * a16[i]

            pltpu.async_copy(zr.at[b], hacc.at[dblk.at[jj]], semsc, add=True)

        pltpu.make_async_copy(zr.at[(SB - 2) % 3], hacc.at[dblk.at[SB - 2]],
                              semsc).wait()
        pltpu.make_async_copy(zr.at[(SB - 1) % 3], hacc.at[dblk.at[SB - 1]],
                              semsc).wait()

    plsc.subcore_barrier()
    pltpu.sync_copy(hacc.at[pl.ds(row0, RPT)],
                    hp_hbm.at[cid, pl.ds(row0, RPT)])


def _combine_body(a_ref, b_ref, o_ref):
    o_ref[...] = a_ref[...] + b_ref[...]


_combine = pl.pallas_call(
    _combine_body,
    grid=(5,),
    in_specs=[
        pl.BlockSpec((2000, D), lambda i: (i, 0)),
        pl.BlockSpec((2000, D), lambda i: (i, 0)),
    ],
    out_specs=pl.BlockSpec((2000, D), lambda i: (i, 0)),
    out_shape=jax.ShapeDtypeStruct((N, D), jnp.float32),
)


def kernel(feature, edge_index, W_fc, W_attn):
    src = edge_index[0].astype(jnp.int32)
    dst = edge_index[1].astype(jnp.int32)
    z, sl, sr = _prep(feature, W_fc, W_attn)
    sl = sl.reshape(N)
    sr = sr.reshape(N)
    zvec = jnp.zeros((NP,), jnp.float32)
    zmat = jnp.zeros((NP, D), jnp.float32)
    eexp, den = _sc_edge(sl, sr, src, dst, zvec)
    _, hp = _sc_agg(z, eexp, den[0], den[1], src, dst, zmat)
    return _combine(hp[0, :N], hp[1, :N])


# speedup vs baseline: 1.1393x; 1.0258x over previous
import functools

import jax
import jax.numpy as jnp
from jax import lax
from jax.experimental import pallas as pl
from jax.experimental.pallas import tpu as pltpu
from jax.experimental.pallas import tpu_sc as plsc

N = 10000
E = 320000
D = 128
NC = 2
NS = 16
NW = NC * NS
L = 16
NP = 10240
RPT = NP // NS
CH = 80
NR = E // NW // CH
SB = 25
NSC = NR // SB

_mesh = plsc.VectorSubcoreMesh(core_axis_name="c", subcore_axis_name="s")


def _prep_body(f_ref, wfc_ref, wat_ref, z_ref, sl_ref, sr_ref):
    z = lax.dot_general(f_ref[...], wfc_ref[...], (((1,), (1,)), ((), ())),
                        preferred_element_type=jnp.float32)
    z_ref[...] = z
    wat = wat_ref[...]
    sl_ref[...] = lax.dot_general(z, wat[:, :D], (((1,), (1,)), ((), ())),
                                  preferred_element_type=jnp.float32)
    sr_ref[...] = lax.dot_general(z, wat[:, D:], (((1,), (1,)), ((), ())),
                                  preferred_element_type=jnp.float32)


_prep = pl.pallas_call(
    _prep_body,
    out_shape=[
        jax.ShapeDtypeStruct((N, D), jnp.float32),
        jax.ShapeDtypeStruct((N, 1), jnp.float32),
        jax.ShapeDtypeStruct((N, 1), jnp.float32),
    ],
)


@functools.partial(
    pl.kernel,
    out_type=[
        jax.ShapeDtypeStruct((E,), jnp.float32),
        jax.ShapeDtypeStruct((NC, NP), jnp.float32),
    ],
    mesh=_mesh,
    scratch_types=[
        pltpu.VMEM((NR, CH), jnp.int32),
        pltpu.VMEM((NR, CH), jnp.int32),
        pltpu.VMEM((NR, CH), jnp.float32),
        pltpu.VMEM((NR, CH), jnp.float32),
        pltpu.VMEM((NR, CH), jnp.float32),
        pltpu.VMEM_SHARED((NP,), jnp.float32),
        pltpu.SemaphoreType.DMA,
        pltpu.SemaphoreType.DMA,
        pltpu.SemaphoreType.DMA,
    ],
)
def _sc_edge(sl_hbm, sr_hbm, src_hbm, dst_hbm, zvec_hbm,
             eexp_hbm, den_hbm,
             sblk, dblk, vl, vr, pblk, dacc, sem1, sem2, semsc):
    cid = lax.axis_index("c")
    sid = lax.axis_index("s")
    row0 = sid * RPT
    wid = sid * NC + cid
    wbase = wid * NR * CH

    @pl.loop(0, NR)
    def _ld(j):
        pltpu.async_copy(src_hbm.at[pl.ds(wbase + j * CH, CH)], sblk.at[j],
                         sem1)
        pltpu.async_copy(dst_hbm.at[pl.ds(wbase + j * CH, CH)], dblk.at[j],
                         sem2)

    pltpu.sync_copy(zvec_hbm.at[pl.ds(row0, RPT)], dacc.at[pl.ds(row0, RPT)])
    plsc.subcore_barrier()

    @pl.loop(0, NR)
    def _ldw(j):
        pltpu.make_async_copy(src_hbm.at[pl.ds(wbase + j * CH, CH)],
                              sblk.at[j], sem1).wait()
        pltpu.make_async_copy(dst_hbm.at[pl.ds(wbase + j * CH, CH)],
                              dblk.at[j], sem2).wait()

    @pl.loop(0, NR)
    def _fire(j):
        pltpu.async_copy(sl_hbm.at[sblk.at[j]], vl.at[j], sem1)
        pltpu.async_copy(sr_hbm.at[dblk.at[j]], vr.at[j], sem2)

    @pl.loop(0, NR)
    def _r(j):
        pltpu.make_async_copy(sl_hbm.at[sblk.at[j]], vl.at[j], sem1).wait()
        pltpu.make_async_copy(sr_hbm.at[dblk.at[j]], vr.at[j], sem2).wait()
        for k in range(CH // L):
            s = pl.ds(k * L, L)
            a = vl[j, s] + vr[j, s]
            e = jnp.where(a >= 0.0, a, a * jnp.float32(0.01))
            pblk[j, s] = jnp.exp(e)
        pltpu.async_copy(pblk.at[j], dacc.at[sblk.at[j]], semsc, add=True)
        pltpu.async_copy(pblk.at[j], eexp_hbm.at[pl.ds(wbase + j * CH, CH)],
                         sem2)

    @pl.loop(0, NR)
    def _dr(j):
        pltpu.make_async_copy(pblk.at[j], dacc.at[sblk.at[j]], semsc).wait()
        pltpu.make_async_copy(pblk.at[j],
                              eexp_hbm.at[pl.ds(wbase + j * CH, CH)],
                              sem2).wait()

    plsc.subcore_barrier()

    @pl.when(sid == 0)
    def _dump():
        pltpu.sync_copy(dacc, den_hbm.at[cid])


def _wscale_body(z_ref, d0_ref, d1_ref, w_ref):
    w_ref[...] = z_ref[...] * (1.0 / (d0_ref[...] + d1_ref[...]))


_wscale = pl.pallas_call(
    _wscale_body,
    grid=(5,),
    in_specs=[
        pl.BlockSpec((2000, D), lambda i: (i, 0)),
        pl.BlockSpec((2000, 1), lambda i: (i, 0)),
        pl.BlockSpec((2000, 1), lambda i: (i, 0)),
    ],
    out_specs=pl.BlockSpec((2000, D), lambda i: (i, 0)),
    out_shape=jax.ShapeDtypeStruct((N, D), jnp.float32),
)


@functools.partial(
    pl.kernel,
    out_type=jax.ShapeDtypeStruct((NC, NP, D), jnp.float32),
    mesh=_mesh,
    scratch_types=[
        pltpu.VMEM((SB * CH,), jnp.int32),
        pltpu.VMEM((SB, CH), jnp.int32),
        pltpu.VMEM((SB * CH,), jnp.float32),
        pltpu.VMEM((3, CH, D), jnp.float32),
        pltpu.VMEM_SHARED((NP, D), jnp.float32),
        pltpu.SemaphoreType.DMA,
        pltpu.SemaphoreType.DMA,
        pltpu.SemaphoreType.DMA,
    ],
)
def _sc_agg(w_hbm, eexp_hbm, src_hbm, dst_hbm, zmat_hbm,
            hp_hbm,
            sblk, dblk, pblk, zr, hacc, semz, semsc, sem1):
    cid = lax.axis_index("c")
    sid = lax.axis_index("s")
    row0 = sid * RPT
    wid = sid * NC + cid
    wbase = wid * NR * CH

    pltpu.sync_copy(zmat_hbm.at[pl.ds(row0, RPT)], hacc.at[pl.ds(row0, RPT)])
    plsc.subcore_barrier()

    @pl.loop(0, NSC)
    def _super(sc):
        base = wbase + sc * SB * CH
        cs = pltpu.async_copy(src_hbm.at[pl.ds(base, SB * CH)], sblk, sem1)
        cp = pltpu.async_copy(eexp_hbm.at[pl.ds(base, SB * CH)], pblk, semz)

        @pl.loop(0, SB)
        def _ldd(jj):
            pltpu.async_copy(dst_hbm.at[pl.ds(base + jj * CH, CH)],
                             dblk.at[jj], semsc)

        cs.wait()
        cp.wait()

        @pl.loop(0, SB)
        def _ldw(jj):
            pltpu.make_async_copy(dst_hbm.at[pl.ds(base + jj * CH, CH)],
                                  dblk.at[jj], semsc).wait()

        pltpu.async_copy(w_hbm.at[sblk.at[pl.ds(0, CH)]], zr.at[0], semz)

        @pl.loop(0, SB)
        def _row(jj):
            b = lax.rem(jj, 3)

            @pl.when(jj >= 2)
            def _drain_scatter():
                bd = lax.rem(jj + 1, 3)
                pltpu.make_async_copy(zr.at[bd], hacc.at[dblk.at[jj - 2]],
                                      semsc).wait()

            @pl.when(jj + 1 < SB)
            def _issue_gather():
                bn = lax.rem(jj + 1, 3)
                pltpu.async_copy(
                    w_hbm.at[sblk.at[pl.ds((jj + 1) * CH, CH)]],
                    zr.at[bn], semz)

            pltpu.make_async_copy(w_hbm.at[sblk.at[pl.ds(jj * CH, CH)]],
                                  zr.at[b], semz).wait()

            for g in range(CH // L):
                a16 = pblk[pl.ds(jj * CH + g * L, L)]
                for i in range(L):
                    r = g * L + i
                    for k in range(D // L):
                        s = pl.ds(k * L, L)
                        zr[b, r, s] = zr[b, r, s] * a16[i]

            pltpu.async_copy(zr.at[b], hacc.at[dblk.at[jj]], semsc, add=True)

        pltpu.make_async_copy(zr.at[(SB - 2) % 3], hacc.at[dblk.at[SB - 2]],
                              semsc).wait()
        pltpu.make_async_copy(zr.at[(SB - 1) % 3], hacc.at[dblk.at[SB - 1]],
                              semsc).wait()

    plsc.subcore_barrier()
    pltpu.sync_copy(hacc.at[pl.ds(row0, RPT)],
                    hp_hbm.at[cid, pl.ds(row0, RPT)])


def _combine_body(a_ref, b_ref, o_ref):
    o_ref[...] = a_ref[...] + b_ref[...]


_combine = pl.pallas_call(
    _combine_body,
    grid=(5,),
    in_specs=[
        pl.BlockSpec((2000, D), lambda i: (i, 0)),
        pl.BlockSpec((2000, D), lambda i: (i, 0)),
    ],
    out_specs=pl.BlockSpec((2000, D), lambda i: (i, 0)),
    out_shape=jax.ShapeDtypeStruct((N, D), jnp.float32),
)


def kernel(feature, edge_index, W_fc, W_attn):
    src = edge_index[0].astype(jnp.int32)
    dst = edge_index[1].astype(jnp.int32)
    z, sl, sr = _prep(feature, W_fc, W_attn)
    sl = sl.reshape(N)
    sr = sr.reshape(N)
    zvec = jnp.zeros((NP,), jnp.float32)
    zmat = jnp.zeros((NP, D), jnp.float32)
    eexp, den = _sc_edge(sl, sr, src, dst, zvec)
    d0 = den[0, :N].reshape(N, 1)
    d1 = den[1, :N].reshape(N, 1)
    w = _wscale(z, d0, d1)
    hp = _sc_agg(w, eexp, src, dst, zmat)
    return _combine(hp[0, :N], hp[1, :N])
